# baseline (speedup 1.0000x reference)
import functools

import jax
import jax.numpy as jnp
from jax import lax
from jax.experimental import pallas as pl
from jax.experimental.pallas import tpu as pltpu
from jax.experimental.pallas import tpu_sc as plsc

_NC, _NS, _L = 2, 16, 16
_T = 8
_NW = _NC * _NS


def _sc_add(emb2d, pos2d, B, S, E):
    s_per_w = S // _NW
    n_t = s_per_w // _T
    n_steps = n_t * B
    n_vec = _T * E // _L
    mesh = plsc.VectorSubcoreMesh(core_axis_name="c", subcore_axis_name="s")

    @functools.partial(
        pl.kernel,
        mesh=mesh,
        out_type=jax.ShapeDtypeStruct((B * S, E), jnp.float32),
        compiler_params=pltpu.CompilerParams(use_tc_tiling_on_sc=True),
        scratch_types=[
            [pltpu.VMEM((_T, E), jnp.float32) for _ in range(4)],
            [pltpu.VMEM((_T, E), jnp.float32) for _ in range(2)],
            [pltpu.SemaphoreType.DMA for _ in range(4)],
            [pltpu.SemaphoreType.DMA for _ in range(4)],
            [pltpu.SemaphoreType.DMA for _ in range(2)],
        ],
    )
    def run(emb_hbm, pos_hbm, out_hbm, bufs, pbufs, lsems, ssems, psems):
        w = lax.axis_index("s") * _NC + lax.axis_index("c")
        base_s = w * s_per_w

        def emb_sl(step):
            t, b = divmod(step, B)
            return pl.ds(b * S + base_s + t * _T, _T)

        pos_d = [
            pltpu.async_copy(pos_hbm.at[pl.ds(base_s + t * _T, _T)],
                             pbufs[t % 2], psems[t % 2])
            for t in range(2)
        ]
        load_d = [None] * n_steps
        store_d = [None] * n_steps
        for s in range(2):
            load_d[s] = pltpu.async_copy(emb_hbm.at[emb_sl(s)],
                                         bufs[s % 4], lsems[s % 4])
        for s in range(n_steps):
            t, b = divmod(s, B)
            if b == 0:
                pos_d[t].wait()
            load_d[s].wait()
            buf, pbuf = bufs[s % 4], pbufs[t % 2]

            @plsc.parallel_loop(0, n_vec, unroll=8)
            def _(i):
                r = i // (E // _L)
                sl = pl.ds((i % (E // _L)) * _L, _L)
                plsc.addupdate(buf.at[r, sl], pbuf[r, sl])

            store_d[s] = pltpu.async_copy(buf, out_hbm.at[emb_sl(s)],
                                          ssems[s % 4])
            if b == B - 1 and t + 2 < n_t:
                pos_d.append(
                    pltpu.async_copy(pos_hbm.at[pl.ds(base_s + (t + 2) * _T, _T)],
                                     pbufs[t % 2], psems[t % 2]))
            ns = s + 2
            if ns < n_steps:
                if ns >= 4:
                    store_d[ns - 4].wait()
                load_d[ns] = pltpu.async_copy(emb_hbm.at[emb_sl(ns)],
                                              bufs[ns % 4], lsems[ns % 4])
        for s in range(n_steps - 4, n_steps):
            store_d[s].wait()

    return run(emb2d, pos2d)


def kernel(embeddings, pos_table):
    B, S, E = embeddings.shape
    out = _sc_add(embeddings.reshape(B * S, E), pos_table, B, S, E)
    return out.reshape(B, S, E)


# speedup vs baseline: 1.0297x; 1.0297x over previous
import functools

import jax
import jax.numpy as jnp
from jax import lax
from jax.experimental import pallas as pl
from jax.experimental.pallas import tpu as pltpu
from jax.experimental.pallas import tpu_sc as plsc

_NC, _NS, _L = 2, 16, 16
_NW = _NC * _NS
_T = 4
_R = 3


def _sc_add(emb2d, pos2d, B, S, E):
    s_per_w = S // _NW
    n_t = s_per_w // _T
    n_vec = _T * E // _L
    mesh = plsc.VectorSubcoreMesh(core_axis_name="c", subcore_axis_name="s")

    @functools.partial(
        pl.kernel,
        mesh=mesh,
        out_type=jax.ShapeDtypeStruct((B * S, E), jnp.float32),
        compiler_params=pltpu.CompilerParams(use_tc_tiling_on_sc=True),
        scratch_types=[
            [[pltpu.VMEM((_T, E), jnp.float32) for _ in range(B)]
             for _ in range(_R)],
            [pltpu.VMEM((_T, E), jnp.float32) for _ in range(_R)],
            [pltpu.SemaphoreType.DMA for _ in range(_R)],
            [pltpu.SemaphoreType.DMA for _ in range(_R)],
            [pltpu.SemaphoreType.DMA for _ in range(_R)],
        ],
    )
    def run(emb_hbm, pos_hbm, out_hbm, ebufs, pbufs, lsems, ssems, psems):
        w = lax.axis_index("s") * _NC + lax.axis_index("c")
        base_s = w * s_per_w

        def row_sl(t, b):
            return pl.ds(b * S + base_s + t * _T, _T)

        def issue_loads(t):
            par = t % _R
            pd = pltpu.async_copy(pos_hbm.at[pl.ds(base_s + t * _T, _T)],
                                  pbufs[par], psems[par])
            eds = [pltpu.async_copy(emb_hbm.at[row_sl(t, b)], ebufs[par][b],
                                    lsems[par])
                   for b in range(B)]
            return pd, eds

        load_d = [None] * n_t
        store_d = [None] * n_t
        for t in range(2):
            load_d[t] = issue_loads(t)
        for t in range(n_t):
            par = t % _R
            pd, eds = load_d[t]
            pd.wait()
            for d in eds:
                d.wait()
            pbuf, grp = pbufs[par], ebufs[par]
            b0, b1, b2, b3 = grp

            @plsc.parallel_loop(0, n_vec, unroll=8)
            def _(i):
                r = i // (E // _L)
                sl = pl.ds((i % (E // _L)) * _L, _L)
                p = pbuf[r, sl]
                plsc.addupdate(b0.at[r, sl], p)
                plsc.addupdate(b1.at[r, sl], p)
                plsc.addupdate(b2.at[r, sl], p)
                plsc.addupdate(b3.at[r, sl], p)

            store_d[t] = [pltpu.async_copy(grp[b], out_hbm.at[row_sl(t, b)],
                                           ssems[par])
                          for b in range(B)]
            nt = t + 2
            if nt < n_t:
                if nt >= _R:
                    for d in store_d[nt - _R]:
                        d.wait()
                load_d[nt] = issue_loads(nt)
        for t in range(n_t - _R, n_t):
            for d in store_d[t]:
                d.wait()

    return run(emb2d, pos2d)


def kernel(embeddings, pos_table):
    B, S, E = embeddings.shape
    out = _sc_add(embeddings.reshape(B * S, E), pos_table, B, S, E)
    return out.reshape(B, S, E)
